# baseline (speedup 1.0000x reference)
import jax
import jax.numpy as jnp
from jax.experimental import pallas as pl
from jax.experimental.pallas import tpu as pltpu

H = 8
D = 64
DA = 128
B = 128


def _qkv_fwd_kernel(wr_b_ref, wr_b1_ref, wc_b_ref, x_ref, w_ref,
                    qk_ref, va_ref, kvf_ref, kv_s):
    b = pl.program_id(0)

    @pl.when(b == 0)
    def _():
        kv_s[...] = jnp.zeros_like(kv_s)

    c = H * D
    acc = jnp.dot(x_ref[...], w_ref[...], preferred_element_type=jnp.float32)
    qkb = jnp.maximum(acc[:, : 2 * c], 0.0).astype(jnp.bfloat16)
    qk_ref[...] = qkb
    one = jnp.ones((B, 1), jnp.bfloat16)
    zer = jnp.zeros((B, DA - D - 1), jnp.bfloat16)
    vb = acc[:, 2 * c :].astype(jnp.bfloat16)
    pieces = []
    for h in range(H):
        pieces += [vb[:, h * D : (h + 1) * D], one, zer]
    vab = jnp.concatenate(pieces, axis=1)
    va_ref[...] = vab
    kvf_ref[0, :, :] = kv_s[...].astype(jnp.bfloat16)
    ws = wr_b_ref[0, :, 0:1]
    wsn = wr_b1_ref[0, :, 0:1]
    prop = ws == wsn
    maskn = wc_b_ref[0, :, :] == wsn
    km = jnp.where(maskn, qkb[:, c:], jnp.bfloat16(0))
    for h in range(H):
        contrib = jax.lax.dot_general(
            km[:, h * D : (h + 1) * D], vab[:, h * DA : (h + 1) * DA],
            (((0,), (0,)), ((), ())),
            preferred_element_type=jnp.float32)
        sl = slice(h * D, (h + 1) * D)
        kv_s[sl, :] = jnp.where(prop, kv_s[sl, :], 0.0) + contrib


def _attn_bwd_kernel(wr_b_ref, wr_bm1_ref, wc_b_ref, qk_ref, va_ref, kvf_ref,
                     wp_ref, bp_ref, o_ref, kvb_s):
    i = pl.program_id(0)

    @pl.when(i == 0)
    def _():
        kvb_s[...] = jnp.zeros_like(kvb_s)

    wrow = wr_b_ref[0, :, :]
    wcol = wc_b_ref[0, :, :]
    ws = wrow[:, 0:1]
    we = wrow[:, B - 1 : B]
    mask2 = wcol == wrow
    c = H * D
    q = qk_ref[:, :c]
    qf = jnp.where(wcol == ws, q, jnp.bfloat16(0))
    qe = jnp.where(wcol == we, q, jnp.bfloat16(0))
    ys = []
    for h in range(H):
        sl = slice(h * D, (h + 1) * D)
        sla = slice(h * DA, (h + 1) * DA)
        qh = q[:, sl]
        kh = qk_ref[:, c + h * D : c + (h + 1) * D]
        a = jax.lax.dot_general(
            qh, kh, (((1,), (1,)), ((), ())),
            preferred_element_type=jnp.float32)
        a = jnp.where(mask2, a, 0.0).astype(jnp.bfloat16)
        yz = jnp.dot(a, va_ref[:, sla], preferred_element_type=jnp.float32)
        yz = yz + jnp.dot(qf[:, sl], kvf_ref[0, sl, :],
                          preferred_element_type=jnp.float32)
        yz = yz + jnp.dot(qe[:, sl], kvb_s[sl, :],
                          preferred_element_type=jnp.float32)
        zr = 1.0 / (yz[:, D : D + 1] + 1e-6)
        ys.append(yz[:, 0:D] * zr)
    y = jnp.concatenate(ys, axis=1)
    o_ref[...] = (jnp.dot(y, wp_ref[...], preferred_element_type=jnp.float32)
                  + bp_ref[0, :])
    wem1 = wr_bm1_ref[0, :, B - 1 : B]
    prop = wem1 == we
    maskm = wcol == wem1
    km = jnp.where(maskm, qk_ref[:, c:], jnp.bfloat16(0))
    for h in range(H):
        contrib = jax.lax.dot_general(
            km[:, h * D : (h + 1) * D], va_ref[:, h * DA : (h + 1) * DA],
            (((0,), (0,)), ((), ())),
            preferred_element_type=jnp.float32)
        sl = slice(h * D, (h + 1) * D)
        kvb_s[sl, :] = jnp.where(prop, kvb_s[sl, :], 0.0) + contrib


def kernel(x, Wqkv, Wproj, bproj, batch_win_inds, offsets, counts):
    del offsets, counts
    n, c = x.shape
    nb = n // B
    win = batch_win_inds.astype(jnp.int32)
    win_row = win.reshape(nb, 1, B)
    win_col = win.reshape(nb, B, 1)
    x = x.astype(jnp.bfloat16)
    Wqkv = Wqkv.astype(jnp.bfloat16)

    qk, va, kvf = pl.pallas_call(
        _qkv_fwd_kernel,
        grid=(nb,),
        in_specs=[
            pl.BlockSpec((1, 1, B), lambda b: (b, 0, 0)),
            pl.BlockSpec((1, 1, B), lambda b: (jnp.minimum(b + 1, nb - 1), 0, 0)),
            pl.BlockSpec((1, B, 1), lambda b: (b, 0, 0)),
            pl.BlockSpec((B, c), lambda b: (b, 0)),
            pl.BlockSpec((c, 3 * c), lambda b: (0, 0)),
        ],
        out_specs=[
            pl.BlockSpec((B, 2 * c), lambda b: (b, 0)),
            pl.BlockSpec((B, 2 * c), lambda b: (b, 0)),
            pl.BlockSpec((1, c, DA), lambda b: (b, 0, 0)),
        ],
        out_shape=[
            jax.ShapeDtypeStruct((n, 2 * c), jnp.bfloat16),
            jax.ShapeDtypeStruct((n, 2 * c), jnp.bfloat16),
            jax.ShapeDtypeStruct((nb, c, DA), jnp.bfloat16),
        ],
        scratch_shapes=[pltpu.VMEM((c, DA), jnp.float32)],
    )(win_row, win_row, win_col, x, Wqkv)

    out = pl.pallas_call(
        _attn_bwd_kernel,
        grid=(nb,),
        in_specs=[
            pl.BlockSpec((1, 1, B), lambda i: (nb - 1 - i, 0, 0)),
            pl.BlockSpec((1, 1, B), lambda i: (jnp.maximum(nb - 2 - i, 0), 0, 0)),
            pl.BlockSpec((1, B, 1), lambda i: (nb - 1 - i, 0, 0)),
            pl.BlockSpec((B, 2 * c), lambda i: (nb - 1 - i, 0)),
            pl.BlockSpec((B, 2 * c), lambda i: (nb - 1 - i, 0)),
            pl.BlockSpec((1, c, DA), lambda i: (nb - 1 - i, 0, 0)),
            pl.BlockSpec((c, c), lambda i: (0, 0)),
            pl.BlockSpec((1, c), lambda i: (0, 0)),
        ],
        out_specs=pl.BlockSpec((B, c), lambda i: (nb - 1 - i, 0)),
        out_shape=jax.ShapeDtypeStruct((n, c), jnp.float32),
        scratch_shapes=[pltpu.VMEM((c, DA), jnp.float32)],
    )(win_row, win_row, win_col, qk, va, kvf, Wproj, bproj.reshape(1, c))
    return out


# speedup vs baseline: 1.4745x; 1.4745x over previous
import jax
import jax.numpy as jnp
from jax.experimental import pallas as pl
from jax.experimental.pallas import tpu as pltpu

H = 8
D = 64
DA = 128
B = 512


def _qkv_fwd_kernel(wr_b_ref, wr_b1_ref, wc_b_ref, x_ref, w_ref,
                    qk_ref, va_ref, kvf_ref, kv_s):
    b = pl.program_id(0)

    @pl.when(b == 0)
    def _():
        kv_s[...] = jnp.zeros_like(kv_s)

    c = H * D
    acc = jnp.dot(x_ref[...], w_ref[...], preferred_element_type=jnp.float32)
    qkb = jnp.maximum(acc[:, : 2 * c], 0.0).astype(jnp.bfloat16)
    qk_ref[...] = qkb
    one = jnp.ones((B, 1), jnp.bfloat16)
    zer = jnp.zeros((B, DA - D - 1), jnp.bfloat16)
    vb = acc[:, 2 * c :].astype(jnp.bfloat16)
    pieces = []
    for h in range(H):
        pieces += [vb[:, h * D : (h + 1) * D], one, zer]
    vab = jnp.concatenate(pieces, axis=1)
    va_ref[...] = vab
    kvf_ref[0, :, :] = kv_s[...].astype(jnp.bfloat16)
    ws = wr_b_ref[0, :, 0:1]
    wsn = wr_b1_ref[0, :, 0:1]
    prop = ws == wsn
    maskn = wc_b_ref[0, :, :] == wsn
    km = jnp.where(maskn, qkb[:, c:], jnp.bfloat16(0))
    for h in range(H):
        contrib = jax.lax.dot_general(
            km[:, h * D : (h + 1) * D], vab[:, h * DA : (h + 1) * DA],
            (((0,), (0,)), ((), ())),
            preferred_element_type=jnp.float32)
        sl = slice(h * D, (h + 1) * D)
        kv_s[sl, :] = jnp.where(prop, kv_s[sl, :], 0.0) + contrib


def _attn_bwd_kernel(wr_b_ref, wr_bm1_ref, wc_b_ref, qk_ref, va_ref, kvf_ref,
                     wp_ref, bp_ref, o_ref, kvb_s):
    i = pl.program_id(0)

    @pl.when(i == 0)
    def _():
        kvb_s[...] = jnp.zeros_like(kvb_s)

    wrow = wr_b_ref[0, :, :]
    wcol = wc_b_ref[0, :, :]
    ws = wrow[:, 0:1]
    we = wrow[:, B - 1 : B]
    mask2 = wcol == wrow
    c = H * D
    q = qk_ref[:, :c]
    qf = jnp.where(wcol == ws, q, jnp.bfloat16(0))
    qe = jnp.where(wcol == we, q, jnp.bfloat16(0))
    ys = []
    for h in range(H):
        sl = slice(h * D, (h + 1) * D)
        sla = slice(h * DA, (h + 1) * DA)
        qh = q[:, sl]
        kh = qk_ref[:, c + h * D : c + (h + 1) * D]
        a = jax.lax.dot_general(
            qh, kh, (((1,), (1,)), ((), ())),
            preferred_element_type=jnp.float32)
        a = jnp.where(mask2, a, 0.0).astype(jnp.bfloat16)
        yz = jnp.dot(a, va_ref[:, sla], preferred_element_type=jnp.float32)
        yz = yz + jnp.dot(qf[:, sl], kvf_ref[0, sl, :],
                          preferred_element_type=jnp.float32)
        yz = yz + jnp.dot(qe[:, sl], kvb_s[sl, :],
                          preferred_element_type=jnp.float32)
        zr = 1.0 / (yz[:, D : D + 1] + 1e-6)
        ys.append(yz[:, 0:D] * zr)
    y = jnp.concatenate(ys, axis=1)
    o_ref[...] = (jnp.dot(y, wp_ref[...], preferred_element_type=jnp.float32)
                  + bp_ref[0, :])
    wem1 = wr_bm1_ref[0, :, B - 1 : B]
    prop = wem1 == we
    maskm = wcol == wem1
    km = jnp.where(maskm, qk_ref[:, c:], jnp.bfloat16(0))
    for h in range(H):
        contrib = jax.lax.dot_general(
            km[:, h * D : (h + 1) * D], va_ref[:, h * DA : (h + 1) * DA],
            (((0,), (0,)), ((), ())),
            preferred_element_type=jnp.float32)
        sl = slice(h * D, (h + 1) * D)
        kvb_s[sl, :] = jnp.where(prop, kvb_s[sl, :], 0.0) + contrib


def kernel(x, Wqkv, Wproj, bproj, batch_win_inds, offsets, counts):
    del offsets, counts
    n, c = x.shape
    nb = n // B
    win = batch_win_inds.astype(jnp.int32)
    win_row = win.reshape(nb, 1, B)
    win_col = win.reshape(nb, B, 1)
    x = x.astype(jnp.bfloat16)
    Wqkv = Wqkv.astype(jnp.bfloat16)

    qk, va, kvf = pl.pallas_call(
        _qkv_fwd_kernel,
        grid=(nb,),
        in_specs=[
            pl.BlockSpec((1, 1, B), lambda b: (b, 0, 0)),
            pl.BlockSpec((1, 1, B), lambda b: (jnp.minimum(b + 1, nb - 1), 0, 0)),
            pl.BlockSpec((1, B, 1), lambda b: (b, 0, 0)),
            pl.BlockSpec((B, c), lambda b: (b, 0)),
            pl.BlockSpec((c, 3 * c), lambda b: (0, 0)),
        ],
        out_specs=[
            pl.BlockSpec((B, 2 * c), lambda b: (b, 0)),
            pl.BlockSpec((B, 2 * c), lambda b: (b, 0)),
            pl.BlockSpec((1, c, DA), lambda b: (b, 0, 0)),
        ],
        out_shape=[
            jax.ShapeDtypeStruct((n, 2 * c), jnp.bfloat16),
            jax.ShapeDtypeStruct((n, 2 * c), jnp.bfloat16),
            jax.ShapeDtypeStruct((nb, c, DA), jnp.bfloat16),
        ],
        scratch_shapes=[pltpu.VMEM((c, DA), jnp.float32)],
    )(win_row, win_row, win_col, x, Wqkv)

    out = pl.pallas_call(
        _attn_bwd_kernel,
        grid=(nb,),
        in_specs=[
            pl.BlockSpec((1, 1, B), lambda i: (nb - 1 - i, 0, 0)),
            pl.BlockSpec((1, 1, B), lambda i: (jnp.maximum(nb - 2 - i, 0), 0, 0)),
            pl.BlockSpec((1, B, 1), lambda i: (nb - 1 - i, 0, 0)),
            pl.BlockSpec((B, 2 * c), lambda i: (nb - 1 - i, 0)),
            pl.BlockSpec((B, 2 * c), lambda i: (nb - 1 - i, 0)),
            pl.BlockSpec((1, c, DA), lambda i: (nb - 1 - i, 0, 0)),
            pl.BlockSpec((c, c), lambda i: (0, 0)),
            pl.BlockSpec((1, c), lambda i: (0, 0)),
        ],
        out_specs=pl.BlockSpec((B, c), lambda i: (nb - 1 - i, 0)),
        out_shape=jax.ShapeDtypeStruct((n, c), jnp.float32),
        scratch_shapes=[pltpu.VMEM((c, DA), jnp.float32)],
    )(win_row, win_row, win_col, qk, va, kvf, Wproj, bproj.reshape(1, c))
    return out
